# Initial kernel scaffold; baseline (speedup 1.0000x reference)
import jax
import jax.numpy as jnp
from jax.experimental import pallas as pl


def kernel(x, edge_index, edge_attr, x_emb1, x_emb2, ee1, ee2, W1, b1, W2, b2, gamma, beta):
    raise NotImplementedError("write your pallas kernel here")


# speedup vs baseline: 3.9218x; 3.9218x over previous
import functools

import jax
import jax.numpy as jnp
from jax import lax
from jax.experimental import pallas as pl
from jax.experimental.pallas import tpu as pltpu
from jax.experimental.pallas import tpu_sc as plsc

NC = 2
NS = 16
NW = NC * NS
K = 80


def _make_sc_scatter(E, NP, D):
  assert E % (NW * K) == 0 and NP % (NS * 128) == 0
  EW = E // NW
  steps = EW // K
  RPT = NP // NS
  WCH = 128
  mesh = plsc.VectorSubcoreMesh(core_axis_name="c", subcore_axis_name="s")
  scratch = [
      pltpu.VMEM_SHARED((NP, D), jnp.float32),
      pltpu.VMEM((K,), jnp.int32),
      pltpu.VMEM((K,), jnp.int32),
      pltpu.VMEM((K,), jnp.int32),
      pltpu.VMEM((K, D), jnp.float32),
      pltpu.VMEM((128, D), jnp.float32),
      pltpu.SemaphoreType.DMA,
  ]
  out_type = jax.ShapeDtypeStruct((NC, NP, D), jnp.float32)

  @functools.partial(pl.kernel, out_type=out_type, mesh=mesh,
                     scratch_types=scratch)
  def k(table, i0, i1, dstv, zeros, out, aggr, sidx, tmp, didx, rows,
        stage, sem):
    cid = lax.axis_index("c")
    sid = lax.axis_index("s")
    wid = sid * NC + cid
    base = wid * EW
    r0 = sid * RPT

    pltpu.sync_copy(zeros.at[pl.ds(0, RPT)], aggr.at[pl.ds(r0, RPT)])
    plsc.subcore_barrier()

    def step(g, carry):
      off = base + g * K
      pltpu.sync_copy(i0.at[pl.ds(off, K)], sidx)
      pltpu.sync_copy(i1.at[pl.ds(off, K)], tmp)
      for i in range(K // 16):
        s = pl.ds(i * 16, 16)
        sidx[s] = sidx[s] + tmp[s] * 3
      pltpu.sync_copy(dstv.at[pl.ds(off, K)], didx)
      pltpu.async_copy(table.at[sidx], rows, sem).wait()
      pltpu.sync_copy(rows, aggr.at[didx], add=True)
      return carry

    lax.fori_loop(0, steps, step, 0)
    plsc.subcore_barrier()

    for j in range(RPT // WCH):
      rr = r0 + j * WCH
      pltpu.sync_copy(aggr.at[pl.ds(rr, WCH)], stage)
      pltpu.sync_copy(stage, out.at[cid, pl.ds(rr, WCH)])

  return k


def _tc_init_body(x0, x1, e1, e2, h):
  n = x0.shape[0]
  lanes = lax.broadcasted_iota(jnp.int32, (n, 128), 1)
  oh1 = (lanes == x0[...]).astype(jnp.float32)
  oh2 = (lanes == x1[...]).astype(jnp.float32)
  h[...] = (jnp.dot(oh1, e1[...], preferred_element_type=jnp.float32, precision=lax.Precision.HIGHEST)
            + jnp.dot(oh2, e2[...], preferred_element_type=jnp.float32, precision=lax.Precision.HIGHEST))


def _tc_combine_body(a, out):
  n = out.shape[0]
  out[...] = a[0, :n] + a[1, :n]


def _tc_layer_body(last, aggr, h, cnt, etab, w1, b1, w2, b2, gm, bt, out):
  n = h.shape[0]
  a = (aggr[0, :n] + aggr[1, :n] + h[...] + etab[12:13, :]
       + jnp.dot(cnt[...], etab[...], preferred_element_type=jnp.float32, precision=lax.Precision.HIGHEST))
  hm = jnp.maximum(
      jnp.dot(a, w1[...], preferred_element_type=jnp.float32) + b1[...], 0.0)
  hn = jnp.dot(hm, w2[...], preferred_element_type=jnp.float32) + b2[...]
  mu = jnp.mean(hn, axis=0, keepdims=True)
  cdev = hn - mu
  var = jnp.mean(cdev * cdev, axis=0, keepdims=True)
  y = cdev / jnp.sqrt(var + 1e-5) * gm[...] + bt[...]
  if not last:
    y = jnp.maximum(y, 0.0)
  out[...] = y


def kernel(x, edge_index, edge_attr, x_emb1, x_emb2, ee1, ee2, W1, b1, W2,
           b2, gamma, beta):
  Nn = x.shape[0]
  E = edge_index.shape[1]
  D = x_emb1.shape[1]
  L = W1.shape[0]
  f32 = jnp.float32

  src = edge_index[0].astype(jnp.int32)
  dst = edge_index[1].astype(jnp.int32)
  ea0 = edge_attr[:, 0].astype(jnp.int32)
  ea1 = edge_attr[:, 1].astype(jnp.int32)

  NP = ((Nn + NS * 128 - 1) // (NS * 128)) * (NS * 128)
  zeros = jnp.zeros((NP // NS, D), f32)
  zeroE = jnp.zeros((E,), jnp.int32)
  eye_tab = jnp.zeros((Nn, D), f32).at[:32, :32].set(jnp.eye(32, dtype=f32))

  e1p = jnp.zeros((128, D), f32).at[: x_emb1.shape[0]].set(x_emb1)
  e2p = jnp.zeros((128, D), f32).at[: x_emb2.shape[0]].set(x_emb2)
  x0 = x[:, 0:1].astype(jnp.int32)
  x1 = x[:, 1:2].astype(jnp.int32)

  kk = jnp.arange(18)
  etabs = ee1[:, kk // 3, :] + ee2[:, kk % 3, :]
  etabs = jnp.concatenate([etabs, jnp.zeros((L, D - 18, D), f32)], axis=1)

  sc_scatter = _make_sc_scatter(E, NP, D)

  tc_init = pl.pallas_call(
      _tc_init_body, out_shape=jax.ShapeDtypeStruct((Nn, D), f32))
  tc_combine = pl.pallas_call(
      _tc_combine_body, out_shape=jax.ShapeDtypeStruct((Nn, D), f32))

  cnt2 = sc_scatter(eye_tab, ea1, ea0, dst, zeros)
  cnt = tc_combine(cnt2)
  h = tc_init(x0, x1, e1p, e2p)

  dep = cnt[0, 0] * 0.0
  for l in range(L):
    aggr2 = sc_scatter(h, src, zeroE, dst, zeros + dep)
    tc_layer = pl.pallas_call(
        functools.partial(_tc_layer_body, l == L - 1),
        out_shape=jax.ShapeDtypeStruct((Nn, D), f32))
    h = tc_layer(aggr2, h, cnt, etabs[l],
                 W1[l], b1[l][None, :], W2[l], b2[l][None, :],
                 gamma[l][None, :], beta[l][None, :])
  return h
